# Initial kernel scaffold; baseline (speedup 1.0000x reference)
import jax
import jax.numpy as jnp
from jax.experimental import pallas as pl


def kernel(V, E, X, W1, b1, W2, b2):
    raise NotImplementedError("write your pallas kernel here")


# speedup vs baseline: 26.4926x; 26.4926x over previous
import functools

import jax
import jax.numpy as jnp
from jax import lax
from jax.experimental import pallas as pl
from jax.experimental.pallas import tpu as pltpu
from jax.experimental.pallas import tpu_sc as plsc

_N = 10000
_NE = 320000
_DIN = 128
_DH = 16
_DOUT = 128

_NPAD = 10240
_STRIPE = _NPAD // 16
_NCORES = 2
_NSUB = 16
_NW = _NCORES * _NSUB
_CHUNK = 128
_K = 79
_NE_PAD = _NW * _K * _CHUNK

_BM = 2048

_vmesh = plsc.VectorSubcoreMesh(core_axis_name="c", subcore_axis_name="s")


@functools.partial(
    pl.kernel,
    mesh=_vmesh,
    out_type=jax.ShapeDtypeStruct((_NCORES, _NPAD, _DH), jnp.float32),
    scratch_types=[
        pltpu.VMEM((_K, _CHUNK), jnp.int32),
        pltpu.VMEM((_K, _CHUNK), jnp.int32),
        pltpu.VMEM((_CHUNK, _DH), jnp.float32),
        pltpu.VMEM((_STRIPE, _DH), jnp.float32),
        pltpu.VMEM_SHARED((_NPAD, _DH), jnp.float32),
        pltpu.SemaphoreType.DMA,
    ],
    compiler_params=pltpu.CompilerParams(use_tc_tiling_on_sc=False),
)
def _prop(y_hbm, src_hbm, dst_hbm, out_hbm,
          src_v, dst_v, rows_v, zero_v, acc_sh, sem):
    c = lax.axis_index("c")
    s = lax.axis_index("s")
    w = c * _NSUB + s

    @pl.loop(0, _STRIPE)
    def _(i):
        zero_v[i, :] = jnp.zeros((_DH,), jnp.float32)

    pltpu.async_copy(src_hbm.at[w], src_v, sem).wait()
    pltpu.async_copy(dst_hbm.at[w], dst_v, sem).wait()
    pltpu.sync_copy(zero_v, acc_sh.at[pl.ds(s * _STRIPE, _STRIPE)])
    plsc.subcore_barrier()

    @pl.loop(0, _K)
    def _(j):
        pltpu.sync_copy(y_hbm.at[src_v.at[j]], rows_v)
        pltpu.sync_copy(rows_v, acc_sh.at[dst_v.at[j]], add=True)

    plsc.subcore_barrier()
    pltpu.sync_copy(acc_sh.at[pl.ds(s * _STRIPE, _STRIPE)],
                    out_hbm.at[c, pl.ds(s * _STRIPE, _STRIPE)])


def _xw1_body(x_ref, w_ref, o_ref):
    o_ref[...] = jnp.dot(x_ref[...], w_ref[...],
                         preferred_element_type=jnp.float32)


def _tc_xw1(xp, w1):
    return pl.pallas_call(
        _xw1_body,
        grid=(_NPAD // _BM,),
        in_specs=[pl.BlockSpec((_BM, _DIN), lambda i: (i, 0)),
                  pl.BlockSpec((_DIN, _DH), lambda i: (0, 0))],
        out_specs=pl.BlockSpec((_BM, _DH), lambda i: (i, 0)),
        out_shape=jax.ShapeDtypeStruct((_NPAD, _DH), jnp.float32),
    )(xp, w1)


def _y1_body(degp_ref, xw_ref, dinv_ref, y1_ref):
    deg = degp_ref[0, :, 0:1] + degp_ref[1, :, 0:1] + 1.0
    dinv = lax.rsqrt(deg)
    dinv_ref[...] = dinv
    y1_ref[...] = xw_ref[...] * dinv


def _tc_y1(degp, xw1):
    return pl.pallas_call(
        _y1_body,
        grid=(_NPAD // _BM,),
        in_specs=[pl.BlockSpec((_NCORES, _BM, _DH), lambda i: (0, i, 0)),
                  pl.BlockSpec((_BM, _DH), lambda i: (i, 0))],
        out_specs=[pl.BlockSpec((_BM, 1), lambda i: (i, 0)),
                   pl.BlockSpec((_BM, _DH), lambda i: (i, 0))],
        out_shape=[jax.ShapeDtypeStruct((_NPAD, 1), jnp.float32),
                   jax.ShapeDtypeStruct((_NPAD, _DH), jnp.float32)],
    )(degp, xw1)


def _h_body(acc_ref, y1_ref, dinv_ref, b1_ref, y2_ref):
    a = acc_ref[0] + acc_ref[1] + y1_ref[...]
    h = jnp.maximum(a * dinv_ref[...] + b1_ref[...], 0.0)
    y2_ref[...] = h * dinv_ref[...]


def _tc_h(acc1, y1, dinv, b1r):
    return pl.pallas_call(
        _h_body,
        grid=(_NPAD // _BM,),
        in_specs=[pl.BlockSpec((_NCORES, _BM, _DH), lambda i: (0, i, 0)),
                  pl.BlockSpec((_BM, _DH), lambda i: (i, 0)),
                  pl.BlockSpec((_BM, 1), lambda i: (i, 0)),
                  pl.BlockSpec((1, _DH), lambda i: (0, 0))],
        out_specs=pl.BlockSpec((_BM, _DH), lambda i: (i, 0)),
        out_shape=jax.ShapeDtypeStruct((_NPAD, _DH), jnp.float32),
    )(acc1, y1, dinv, b1r)


def _out_body(acc_ref, y2_ref, dinv_ref, w2_ref, b2_ref, o_ref):
    p = (acc_ref[0] + acc_ref[1] + y2_ref[...]) * dinv_ref[...]
    o_ref[...] = jnp.dot(p, w2_ref[...],
                         preferred_element_type=jnp.float32) + b2_ref[...]


def _tc_out(acc2, y2, dinv, w2, b2r):
    return pl.pallas_call(
        _out_body,
        grid=(_NPAD // _BM,),
        in_specs=[pl.BlockSpec((_NCORES, _BM, _DH), lambda i: (0, i, 0)),
                  pl.BlockSpec((_BM, _DH), lambda i: (i, 0)),
                  pl.BlockSpec((_BM, 1), lambda i: (i, 0)),
                  pl.BlockSpec((_DH, _DOUT), lambda i: (0, 0)),
                  pl.BlockSpec((1, _DOUT), lambda i: (0, 0))],
        out_specs=pl.BlockSpec((_BM, _DOUT), lambda i: (i, 0)),
        out_shape=jax.ShapeDtypeStruct((_NPAD, _DOUT), jnp.float32),
    )(acc2, y2, dinv, w2, b2r)


def kernel(V, E, X, W1, b1, W2, b2):
    del V
    src = E[0]
    dst = E[1]
    fill = jnp.full((_NE_PAD - _NE,), _N, jnp.int32)
    srcp = jnp.concatenate([src, fill]).reshape(_NW, _K, _CHUNK)
    dstp = jnp.concatenate([dst, fill]).reshape(_NW, _K, _CHUNK)
    xp = jnp.pad(X, ((0, _NPAD - _N), (0, 0)))
    ones = jnp.ones((_NPAD, _DH), jnp.float32)

    degp = _prop(ones, dstp, dstp)
    xw1 = _tc_xw1(xp, W1)
    dinv, y1 = _tc_y1(degp, xw1)
    acc1 = _prop(y1, srcp, dstp)
    y2 = _tc_h(acc1, y1, dinv, b1.reshape(1, _DH))
    acc2 = _prop(y2, srcp, dstp)
    out = _tc_out(acc2, y2, dinv, W2, b2.reshape(1, _DOUT))
    return out[:_N]


# speedup vs baseline: 28.8285x; 1.0882x over previous
import functools

import jax
import jax.numpy as jnp
from jax import lax
from jax.experimental import pallas as pl
from jax.experimental.pallas import tpu as pltpu
from jax.experimental.pallas import tpu_sc as plsc

_N = 10000
_NE = 320000
_DIN = 128
_DH = 16
_DOUT = 128

_NPAD = 10240
_STRIPE = _NPAD // 16
_NCORES = 2
_NSUB = 16
_NW = _NCORES * _NSUB
_CHUNK = 128
_K = 80
_KBUF = _K + 2
_NE_PAD = _NW * _KBUF * _CHUNK

_BM = 2048

_vmesh = plsc.VectorSubcoreMesh(core_axis_name="c", subcore_axis_name="s")


@functools.partial(
    pl.kernel,
    mesh=_vmesh,
    out_type=jax.ShapeDtypeStruct((_NCORES, _NPAD, _DH), jnp.float32),
    scratch_types=[
        pltpu.VMEM((_KBUF, _CHUNK), jnp.int32),
        pltpu.VMEM((_K, _CHUNK), jnp.int32),
        pltpu.VMEM((_CHUNK, _DH), jnp.float32),
        pltpu.VMEM((_CHUNK, _DH), jnp.float32),
        pltpu.VMEM((_STRIPE, _DH), jnp.float32),
        pltpu.VMEM_SHARED((_NPAD, _DH), jnp.float32),
        pltpu.SemaphoreType.DMA,
        pltpu.SemaphoreType.DMA,
    ],
    compiler_params=pltpu.CompilerParams(use_tc_tiling_on_sc=False),
)
def _prop(y_hbm, src_hbm, dst_hbm, out_hbm,
          src_v, dst_v, rows0_v, rows1_v, zero_v, acc_sh, sem0, sem1):
    c = lax.axis_index("c")
    s = lax.axis_index("s")
    w = c * _NSUB + s

    @pl.loop(0, _STRIPE)
    def _(i):
        zero_v[i, :] = jnp.zeros((_DH,), jnp.float32)

    pltpu.async_copy(src_hbm.at[w], src_v, sem0).wait()
    pltpu.async_copy(dst_hbm.at[w, pl.ds(0, _K)], dst_v, sem0).wait()
    pltpu.sync_copy(zero_v, acc_sh.at[pl.ds(s * _STRIPE, _STRIPE)])
    plsc.subcore_barrier()

    pltpu.async_copy(y_hbm.at[src_v.at[0]], rows0_v, sem0)
    pltpu.async_copy(y_hbm.at[src_v.at[1]], rows1_v, sem1)

    @pl.loop(0, _K, step=2)
    def _(j):
        pltpu.make_async_copy(y_hbm.at[src_v.at[j]], rows0_v, sem0).wait()
        pltpu.sync_copy(rows0_v, acc_sh.at[dst_v.at[j]], add=True)
        pltpu.async_copy(y_hbm.at[src_v.at[j + 2]], rows0_v, sem0)
        pltpu.make_async_copy(y_hbm.at[src_v.at[j + 1]], rows1_v, sem1).wait()
        pltpu.sync_copy(rows1_v, acc_sh.at[dst_v.at[j + 1]], add=True)
        pltpu.async_copy(y_hbm.at[src_v.at[j + 3]], rows1_v, sem1)

    pltpu.make_async_copy(y_hbm.at[src_v.at[_K]], rows0_v, sem0).wait()
    pltpu.make_async_copy(y_hbm.at[src_v.at[_K + 1]], rows1_v, sem1).wait()

    plsc.subcore_barrier()
    pltpu.sync_copy(acc_sh.at[pl.ds(s * _STRIPE, _STRIPE)],
                    out_hbm.at[c, pl.ds(s * _STRIPE, _STRIPE)])


@functools.partial(
    pl.kernel,
    mesh=_vmesh,
    out_type=jax.ShapeDtypeStruct((_NCORES, _NPAD, _DH), jnp.float32),
    scratch_types=[
        pltpu.VMEM((_K, _CHUNK), jnp.int32),
        pltpu.VMEM((_CHUNK, _DH), jnp.float32),
        pltpu.VMEM((_STRIPE, _DH), jnp.float32),
        pltpu.VMEM_SHARED((_NPAD, _DH), jnp.float32),
        pltpu.SemaphoreType.DMA,
    ],
    compiler_params=pltpu.CompilerParams(use_tc_tiling_on_sc=False),
)
def _deg(dst_hbm, out_hbm, dst_v, ones_v, zero_v, acc_sh, sem):
    c = lax.axis_index("c")
    s = lax.axis_index("s")
    w = c * _NSUB + s

    @pl.loop(0, _STRIPE)
    def _(i):
        zero_v[i, :] = jnp.zeros((_DH,), jnp.float32)

    @pl.loop(0, _CHUNK)
    def _(i):
        ones_v[i, :] = jnp.full((_DH,), 1.0, jnp.float32)

    pltpu.async_copy(dst_hbm.at[w, pl.ds(0, _K)], dst_v, sem).wait()
    pltpu.sync_copy(zero_v, acc_sh.at[pl.ds(s * _STRIPE, _STRIPE)])
    plsc.subcore_barrier()

    @pl.loop(0, _K)
    def _(j):
        pltpu.sync_copy(ones_v, acc_sh.at[dst_v.at[j]], add=True)

    plsc.subcore_barrier()
    pltpu.sync_copy(acc_sh.at[pl.ds(s * _STRIPE, _STRIPE)],
                    out_hbm.at[c, pl.ds(s * _STRIPE, _STRIPE)])


def _xw1_body(x_ref, w_ref, o_ref):
    o_ref[...] = jnp.dot(x_ref[...], w_ref[...],
                         preferred_element_type=jnp.float32)


def _tc_xw1(xp, w1):
    return pl.pallas_call(
        _xw1_body,
        grid=(_NPAD // _BM,),
        in_specs=[pl.BlockSpec((_BM, _DIN), lambda i: (i, 0)),
                  pl.BlockSpec((_DIN, _DH), lambda i: (0, 0))],
        out_specs=pl.BlockSpec((_BM, _DH), lambda i: (i, 0)),
        out_shape=jax.ShapeDtypeStruct((_NPAD, _DH), jnp.float32),
    )(xp, w1)


def _y1_body(degp_ref, xw_ref, dinv_ref, y1_ref):
    deg = degp_ref[0, :, 0:1] + degp_ref[1, :, 0:1] + 1.0
    dinv = lax.rsqrt(deg)
    dinv_ref[...] = dinv
    y1_ref[...] = xw_ref[...] * dinv


def _tc_y1(degp, xw1):
    return pl.pallas_call(
        _y1_body,
        grid=(_NPAD // _BM,),
        in_specs=[pl.BlockSpec((_NCORES, _BM, _DH), lambda i: (0, i, 0)),
                  pl.BlockSpec((_BM, _DH), lambda i: (i, 0))],
        out_specs=[pl.BlockSpec((_BM, 1), lambda i: (i, 0)),
                   pl.BlockSpec((_BM, _DH), lambda i: (i, 0))],
        out_shape=[jax.ShapeDtypeStruct((_NPAD, 1), jnp.float32),
                   jax.ShapeDtypeStruct((_NPAD, _DH), jnp.float32)],
    )(degp, xw1)


def _h_body(acc_ref, y1_ref, dinv_ref, b1_ref, y2_ref):
    a = acc_ref[0] + acc_ref[1] + y1_ref[...]
    h = jnp.maximum(a * dinv_ref[...] + b1_ref[...], 0.0)
    y2_ref[...] = h * dinv_ref[...]


def _tc_h(acc1, y1, dinv, b1r):
    return pl.pallas_call(
        _h_body,
        grid=(_NPAD // _BM,),
        in_specs=[pl.BlockSpec((_NCORES, _BM, _DH), lambda i: (0, i, 0)),
                  pl.BlockSpec((_BM, _DH), lambda i: (i, 0)),
                  pl.BlockSpec((_BM, 1), lambda i: (i, 0)),
                  pl.BlockSpec((1, _DH), lambda i: (0, 0))],
        out_specs=pl.BlockSpec((_BM, _DH), lambda i: (i, 0)),
        out_shape=jax.ShapeDtypeStruct((_NPAD, _DH), jnp.float32),
    )(acc1, y1, dinv, b1r)


def _out_body(acc_ref, y2_ref, dinv_ref, w2_ref, b2_ref, o_ref):
    p = (acc_ref[0] + acc_ref[1] + y2_ref[...]) * dinv_ref[...]
    o_ref[...] = jnp.dot(p, w2_ref[...],
                         preferred_element_type=jnp.float32) + b2_ref[...]


def _tc_out(acc2, y2, dinv, w2, b2r):
    return pl.pallas_call(
        _out_body,
        grid=(_NPAD // _BM,),
        in_specs=[pl.BlockSpec((_NCORES, _BM, _DH), lambda i: (0, i, 0)),
                  pl.BlockSpec((_BM, _DH), lambda i: (i, 0)),
                  pl.BlockSpec((_BM, 1), lambda i: (i, 0)),
                  pl.BlockSpec((_DH, _DOUT), lambda i: (0, 0)),
                  pl.BlockSpec((1, _DOUT), lambda i: (0, 0))],
        out_specs=pl.BlockSpec((_BM, _DOUT), lambda i: (i, 0)),
        out_shape=jax.ShapeDtypeStruct((_NPAD, _DOUT), jnp.float32),
    )(acc2, y2, dinv, w2, b2r)


def kernel(V, E, X, W1, b1, W2, b2):
    del V
    src = E[0]
    dst = E[1]
    fill = jnp.full((_NW * _K * _CHUNK - _NE,), _N, jnp.int32)
    dstp = jnp.concatenate([dst, fill]).reshape(_NW, _K, _CHUNK)
    srcp = jnp.concatenate(
        [jnp.concatenate([src, fill]).reshape(_NW, _K, _CHUNK),
         jnp.full((_NW, _KBUF - _K, _CHUNK), _N, jnp.int32)], axis=1)
    xp = jnp.pad(X, ((0, _NPAD - _N), (0, 0)))

    degp = _deg(dstp)
    xw1 = _tc_xw1(xp, W1)
    dinv, y1 = _tc_y1(degp, xw1)
    acc1 = _prop(y1, srcp, dstp)
    y2 = _tc_h(acc1, y1, dinv, b1.reshape(1, _DH))
    acc2 = _prop(y2, srcp, dstp)
    out = _tc_out(acc2, y2, dinv, W2, b2.reshape(1, _DOUT))
    return out[:_N]
